# baseline (speedup 1.0000x reference)
import functools

import jax
import jax.numpy as jnp
from jax import lax
from jax.experimental import pallas as pl
from jax.experimental.pallas import tpu as pltpu
from jax.experimental.pallas import tpu_sc as plsc

D_MODEL = 1024
SCALE = 32.0
NC, NS = 2, 16
NW = NC * NS
LANES = 16
CHUNK = 8
NBUF = 8
AHEAD = 6


@functools.cache
def _emb_call(B):
    b_per_w = B // NW
    n_chunks = b_per_w // CHUNK
    assert n_chunks % NBUF == 0 and n_chunks >= NBUF
    mesh = plsc.VectorSubcoreMesh(core_axis_name="c", subcore_axis_name="s")

    @functools.partial(
        pl.kernel,
        out_type=jax.ShapeDtypeStruct((B, D_MODEL), jnp.float32),
        mesh=mesh,
        scratch_types=[
            pltpu.VMEM((b_per_w,), jnp.int32),
            pltpu.VMEM((NBUF, CHUNK, D_MODEL), jnp.float32),
            pltpu.SemaphoreType.DMA((NBUF,)),
            pltpu.SemaphoreType.DMA((NBUF,)),
        ],
    )
    def k(idx_hbm, table_hbm, out_hbm, idx_v, rows, in_sems, out_sems):
        wid = lax.axis_index("s") * NC + lax.axis_index("c")
        base = wid * b_per_w
        pltpu.sync_copy(idx_hbm.at[pl.ds(base, b_per_w)], idx_v)

        def gather_start(i, b):
            pltpu.async_copy(
                table_hbm.at[idx_v.at[pl.ds(i * CHUNK, CHUNK)]],
                rows.at[b],
                in_sems.at[b],
            )

        def gather_wait(b):
            pltpu.make_async_copy(
                table_hbm.at[idx_v.at[pl.ds(0, CHUNK)]],
                rows.at[b],
                in_sems.at[b],
            ).wait()

        def scatter_start(i, b):
            pltpu.async_copy(
                rows.at[b],
                out_hbm.at[pl.ds(base + i * CHUNK, CHUNK)],
                out_sems.at[b],
            )

        def scatter_wait(b):
            pltpu.make_async_copy(
                rows.at[b], out_hbm.at[pl.ds(0, CHUNK)], out_sems.at[b]
            ).wait()

        for b in range(AHEAD):
            gather_start(b, b)

        def step(t, carry):
            i0 = t * NBUF
            for b in range(NBUF):
                i = i0 + b
                nxt = (b + AHEAD) % NBUF

                @pl.when(i + AHEAD < n_chunks)
                def _():
                    @pl.when(i >= NBUF - AHEAD)
                    def _():
                        scatter_wait(nxt)

                    gather_start(i + AHEAD, nxt)

                gather_wait(b)

                def row_body(r, c):
                    for j in range(D_MODEL // LANES):
                        sl = pl.ds(j * LANES, LANES)
                        rows[b, r, sl] = rows[b, r, sl] * SCALE
                    return c

                lax.fori_loop(0, CHUNK, row_body, 0)

                scatter_start(i, b)
            return carry

        lax.fori_loop(0, n_chunks // NBUF, step, 0)
        for b in range(NBUF):
            scatter_wait(b)

    return k


def _burn_body(o_ref):
    def body(i, c):
        return c * 1.000001

    o_ref[...] = lax.fori_loop(0, 60000, body, jnp.ones((8, 128), jnp.float32))


def _tc_burn():
    return pl.pallas_call(
        _burn_body,
        out_shape=jax.ShapeDtypeStruct((8, 128), jnp.float32),
        compiler_params=pltpu.CompilerParams(has_side_effects=True),
    )()


def kernel(x, table):
    b, s = x.shape
    flat = x.reshape(b * s)
    out = _emb_call(b * s)(flat, table)
    _tc_burn()
    return out.reshape(b, s, D_MODEL)


# speedup vs baseline: 3.1853x; 3.1853x over previous
import functools

import jax
import jax.numpy as jnp
from jax import lax
from jax.experimental import pallas as pl
from jax.experimental.pallas import tpu as pltpu
from jax.experimental.pallas import tpu_sc as plsc

D_MODEL = 1024
SCALE = 32.0
NC, NS = 2, 16
NW = NC * NS
LANES = 16
CHUNK = 8
NBUF = 8
AHEAD = 7


@functools.cache
def _emb_call(B):
    b_per_w = B // NW
    n_chunks = b_per_w // CHUNK
    assert n_chunks % NBUF == 0 and n_chunks >= NBUF
    mesh = plsc.VectorSubcoreMesh(core_axis_name="c", subcore_axis_name="s")

    @functools.partial(
        pl.kernel,
        out_type=jax.ShapeDtypeStruct((B, D_MODEL), jnp.float32),
        mesh=mesh,
        scratch_types=[
            pltpu.VMEM((b_per_w,), jnp.int32),
            pltpu.VMEM((NBUF, CHUNK, D_MODEL), jnp.float32),
            pltpu.SemaphoreType.DMA((NBUF,)),
            pltpu.SemaphoreType.DMA((NBUF,)),
        ],
    )
    def k(idx_hbm, table_hbm, out_hbm, idx_v, rows, in_sems, out_sems):
        wid = lax.axis_index("s") * NC + lax.axis_index("c")
        base = wid * b_per_w
        pltpu.sync_copy(idx_hbm.at[pl.ds(base, b_per_w)], idx_v)

        def gather_start(i, b):
            pltpu.async_copy(
                table_hbm.at[idx_v.at[pl.ds(i * CHUNK, CHUNK)]],
                rows.at[b],
                in_sems.at[b],
            )

        def gather_wait(b):
            pltpu.make_async_copy(
                table_hbm.at[idx_v.at[pl.ds(0, CHUNK)]],
                rows.at[b],
                in_sems.at[b],
            ).wait()

        def scatter_start(i, b):
            pltpu.async_copy(
                rows.at[b],
                out_hbm.at[pl.ds(base + i * CHUNK, CHUNK)],
                out_sems.at[b],
            )

        def scatter_wait(b):
            pltpu.make_async_copy(
                rows.at[b], out_hbm.at[pl.ds(0, CHUNK)], out_sems.at[b]
            ).wait()

        for b in range(AHEAD):
            gather_start(b, b)

        def step(t, carry):
            i0 = t * NBUF
            for b in range(NBUF):
                i = i0 + b
                nxt = (b + AHEAD) % NBUF

                @pl.when(i + AHEAD < n_chunks)
                def _():
                    @pl.when(i >= NBUF - AHEAD)
                    def _():
                        scatter_wait(nxt)

                    gather_start(i + AHEAD, nxt)

                gather_wait(b)

                def row_body(r, c):
                    for j in range(D_MODEL // LANES):
                        sl = pl.ds(j * LANES, LANES)
                        rows[b, r, sl] = rows[b, r, sl] * SCALE
                    return c

                lax.fori_loop(0, CHUNK, row_body, 0)

                scatter_start(i, b)
            return carry

        lax.fori_loop(0, n_chunks // NBUF, step, 0)
        for b in range(NBUF):
            scatter_wait(b)

    return k


def kernel(x, table):
    b, s = x.shape
    flat = x.reshape(b * s)
    out = _emb_call(b * s)(flat, table)
    return out.reshape(b, s, D_MODEL)


# speedup vs baseline: 3.1992x; 1.0044x over previous
import functools

import jax
import jax.numpy as jnp
from jax import lax
from jax.experimental import pallas as pl
from jax.experimental.pallas import tpu as pltpu
from jax.experimental.pallas import tpu_sc as plsc

D_MODEL = 1024
SCALE = 32.0
NC, NS = 2, 16
NW = NC * NS
LANES = 16
CHUNK = 8
NBUF = 8
AHEAD = 6


@functools.cache
def _emb_call(B):
    b_per_w = B // NW
    n_chunks = b_per_w // CHUNK
    assert n_chunks % NBUF == 0 and n_chunks >= NBUF
    mesh = plsc.VectorSubcoreMesh(core_axis_name="c", subcore_axis_name="s")

    @functools.partial(
        pl.kernel,
        out_type=jax.ShapeDtypeStruct((B, D_MODEL), jnp.float32),
        mesh=mesh,
        scratch_types=[
            pltpu.VMEM((b_per_w,), jnp.int32),
            pltpu.VMEM((NBUF, CHUNK, D_MODEL), jnp.float32),
            pltpu.SemaphoreType.DMA((NBUF,)),
            pltpu.SemaphoreType.DMA((NBUF,)),
        ],
    )
    def k(idx_hbm, table_hbm, out_hbm, idx_v, rows, in_sems, out_sems):
        wid = lax.axis_index("s") * NC + lax.axis_index("c")
        base = wid * b_per_w
        pltpu.sync_copy(idx_hbm.at[pl.ds(base, b_per_w)], idx_v)

        def gather_start(i, b):
            pltpu.async_copy(
                table_hbm.at[idx_v.at[pl.ds(i * CHUNK, CHUNK)]],
                rows.at[b],
                in_sems.at[b],
            )

        def gather_wait(b):
            pltpu.make_async_copy(
                table_hbm.at[idx_v.at[pl.ds(0, CHUNK)]],
                rows.at[b],
                in_sems.at[b],
            ).wait()

        def scatter_start(i, b):
            pltpu.async_copy(
                rows.at[b],
                out_hbm.at[pl.ds(base + i * CHUNK, CHUNK)],
                out_sems.at[b],
            )

        def scatter_wait(b):
            pltpu.make_async_copy(
                rows.at[b], out_hbm.at[pl.ds(0, CHUNK)], out_sems.at[b]
            ).wait()

        for b in range(AHEAD):
            gather_start(b, b)

        def step(t, carry):
            i0 = t * NBUF
            for b in range(NBUF):
                i = i0 + b
                nxt = (b + AHEAD) % NBUF

                @pl.when(i + AHEAD < n_chunks)
                def _():
                    @pl.when(i >= NBUF - AHEAD)
                    def _():
                        scatter_wait(nxt)

                    gather_start(i + AHEAD, nxt)

                gather_wait(b)

                def row_body(r, c):
                    for j in range(D_MODEL // LANES):
                        sl = pl.ds(j * LANES, LANES)
                        rows[b, r, sl] = rows[b, r, sl] * SCALE
                    return c

                lax.fori_loop(0, CHUNK, row_body, 0)

                scatter_start(i, b)
            return carry

        lax.fori_loop(0, n_chunks // NBUF, step, 0)
        for b in range(NBUF):
            scatter_wait(b)

    return k


def kernel(x, table):
    b, s = x.shape
    flat = x.reshape(b * s)
    out = _emb_call(b * s)(flat, table)
    return out.reshape(b, s, D_MODEL)
